# Initial kernel scaffold; baseline (speedup 1.0000x reference)
import functools

import jax
import jax.numpy as jnp
from jax import lax
from jax.experimental import pallas as pl
from jax.experimental.pallas import tpu as pltpu
from jax.experimental.pallas import tpu_sc as plsc

_BINS = (-3, -2, -1, 0, 1, 2, 3)
_D = 20
_L = 16
_NW = 32
_C = 1024


def _body(len_hbm, w_hbm, out_hbm, tab_v, len_v, idx_v, out_v):
    n = len_hbm.shape[0]
    per_w = n // _NW
    wid = lax.axis_index("s") * 2 + lax.axis_index("c")
    base = wid * per_w

    pltpu.sync_copy(w_hbm, tab_v)

    def chunk(ci, _):
        row0 = base + ci * _C
        pltpu.sync_copy(len_hbm.at[pl.ds(row0, _C)], len_v)

        def bucketize(g, _):
            l = len_v[pl.ds(g * _L, _L)]
            acc = jnp.zeros((_L,), jnp.int32)
            for b in _BINS:
                acc = acc + (l >= b).astype(jnp.int32)
            idx_v[pl.ds(g * _L, _L)] = acc
            return 0

        lax.fori_loop(0, _C // _L, bucketize, 0, unroll=4)

        def emit(g, _):
            p = g * _L + lax.iota(jnp.int32, _L)
            i = p // _D
            j = p - i * _D
            e = plsc.load_gather(idx_v, [i])
            v = plsc.load_gather(tab_v, [e, j])
            out_v[pl.ds(g * _L, _L)] = v
            return 0

        lax.fori_loop(0, _C * _D // _L, emit, 0, unroll=4)

        pltpu.sync_copy(out_v, out_hbm.at[pl.ds(row0 * _D, _C * _D)])
        return 0

    lax.fori_loop(0, per_w // _C, chunk, 0)


def kernel(lengths, W):
    n = lengths.shape[0]
    lengths = lengths.astype(jnp.int32)
    W = W.astype(jnp.float32)

    mesh = plsc.VectorSubcoreMesh(core_axis_name="c", subcore_axis_name="s")
    out = pl.kernel(
        _body,
        out_type=jax.ShapeDtypeStruct((n * _D,), jnp.float32),
        mesh=mesh,
        scratch_types=[
            pltpu.VMEM((8, _D), jnp.float32),
            pltpu.VMEM((_C,), jnp.int32),
            pltpu.VMEM((_C,), jnp.int32),
            pltpu.VMEM((_C * _D,), jnp.float32)
        ],
    )(lengths, W)
    return out.reshape(n, _D)


# speedup vs baseline: 2.9013x; 2.9013x over previous
import jax
import jax.numpy as jnp
from jax import lax
from jax.experimental import pallas as pl
from jax.experimental.pallas import tpu as pltpu
from jax.experimental.pallas import tpu_sc as plsc

_D = 20
_L = 16
_NW = 32
_C = 1024


def _body(len_hbm, w_hbm, out_hbm, tab_v, len_v, out_v):
    n = len_hbm.shape[0]
    per_w = n // _NW
    wid = lax.axis_index("s") * 2 + lax.axis_index("c")
    base = wid * per_w

    pltpu.sync_copy(w_hbm, tab_v)

    def chunk(ci, _):
        row0 = base + ci * _C
        pltpu.sync_copy(len_hbm.at[pl.ds(row0, _C)], len_v)

        def emit(g, _):
            p = g * _L + lax.iota(jnp.int32, _L)
            i = lax.div(p, jnp.int32(_D))
            j = p - i * _D
            l = plsc.load_gather(len_v, [i])
            e = jnp.minimum(jnp.maximum(l + 4, 0), 7)
            out_v[pl.ds(g * _L, _L)] = plsc.load_gather(tab_v, [e, j])
            return 0

        lax.fori_loop(0, _C * _D // _L, emit, 0, unroll=4)

        pltpu.sync_copy(out_v, out_hbm.at[pl.ds(row0 * _D, _C * _D)])
        return 0

    lax.fori_loop(0, per_w // _C, chunk, 0)


def kernel(lengths, W):
    n = lengths.shape[0]
    lengths = lengths.astype(jnp.int32)
    W = W.astype(jnp.float32)

    mesh = plsc.VectorSubcoreMesh(core_axis_name="c", subcore_axis_name="s")
    out = pl.kernel(
        _body,
        out_type=jax.ShapeDtypeStruct((n * _D,), jnp.float32),
        mesh=mesh,
        compiler_params=pltpu.CompilerParams(needs_layout_passes=False),
        scratch_types=[
            pltpu.VMEM((8, _D), jnp.float32),
            pltpu.VMEM((_C,), jnp.int32),
            pltpu.VMEM((_C * _D,), jnp.float32),
        ],
    )(lengths, W)
    return out.reshape(n, _D)


# speedup vs baseline: 3.9943x; 1.3767x over previous
import jax
import jax.numpy as jnp
from jax import lax
from jax.experimental import pallas as pl
from jax.experimental.pallas import tpu as pltpu
from jax.experimental.pallas import tpu_sc as plsc

_D = 20
_L = 16
_NW = 32
_C = 1024


def _body(len_hbm, w_hbm, out_hbm, tab_v, len_v, out_v):
    n = len_hbm.shape[0]
    per_w = n // _NW
    wid = lax.axis_index("s") * 2 + lax.axis_index("c")
    base = wid * per_w

    pltpu.sync_copy(w_hbm, tab_v)

    def chunk(ci, _):
        row0 = base + ci * _C
        pltpu.sync_copy(len_hbm.at[pl.ds(row0, _C)], len_v)

        @plsc.parallel_loop(0, _C * _D, step=_L, unroll=8)
        def emit(p0):
            p = p0 + lax.iota(jnp.int32, _L)
            i = lax.div(p, jnp.int32(_D))
            j = p - i * _D
            l = plsc.load_gather(len_v, [i])
            e = jnp.minimum(jnp.maximum(l + 4, 0), 7)
            out_v[pl.ds(p0, _L)] = plsc.load_gather(tab_v, [e, j])

        pltpu.sync_copy(out_v, out_hbm.at[pl.ds(row0 * _D, _C * _D)])
        return 0

    lax.fori_loop(0, per_w // _C, chunk, 0)


def kernel(lengths, W):
    n = lengths.shape[0]
    lengths = lengths.astype(jnp.int32)
    W = W.astype(jnp.float32)

    mesh = plsc.VectorSubcoreMesh(core_axis_name="c", subcore_axis_name="s")
    out = pl.kernel(
        _body,
        out_type=jax.ShapeDtypeStruct((n * _D,), jnp.float32),
        mesh=mesh,
        compiler_params=pltpu.CompilerParams(needs_layout_passes=False),
        scratch_types=[
            pltpu.VMEM((8, _D), jnp.float32),
            pltpu.VMEM((_C,), jnp.int32),
            pltpu.VMEM((_C * _D,), jnp.float32),
        ],
    )(lengths, W)
    return out.reshape(n, _D)
